# baseline (speedup 1.0000x reference)
import functools

import numpy as np
import jax
import jax.numpy as jnp
from jax import lax
from jax.experimental import pallas as pl
from jax.experimental.pallas import tpu as pltpu
from jax.experimental.pallas import tpu_sc as plsc

_L = 68
_NCOL = _L * (_L - 1) // 2
_B = 16384
_NW = 32
_ROWS_PER_W = _B // _NW
_R = 8
_NCHUNK = _ROWS_PER_W // _R
_NPAIR = _NCHUNK // 2
_JS = (0, 16, 32, 48, 64)
_NT = len(_JS)
_JW = _NT * 16
_TAB = _L * _JW


def _tables():
    i = np.arange(_L)[:, None]
    jpos = np.concatenate([np.arange(js, js + 16) for js in _JS])[None, :]
    tri_i = (i * (i - 1)) // 2
    tri_j = (jpos * (jpos - 1)) // 2
    idx = np.where(i > jpos, tri_i + jpos, np.minimum(tri_j + i, _NCOL - 1))
    idx = np.where(i == jpos, 0, idx)
    sgn = np.where(i > jpos, 1.0, np.where(i < jpos, -1.0, 0.0))
    sgn = np.where(jpos >= _L, 0.0, sgn)
    return (jnp.asarray(idx.reshape(-1), dtype=jnp.int32),
            jnp.asarray(sgn.reshape(-1), dtype=jnp.float32))


_H = _R // 2


def _body(x_hbm, idx_hbm, sgn_hbm, out_hbm,
          idx_v, sgn_v, x0, x1, o0, o1, si0, si1, so0, so1):
    wid = lax.axis_index("s") * 2 + lax.axis_index("c")
    base = wid * _ROWS_PER_W
    pltpu.sync_copy(idx_hbm, idx_v)
    pltpu.sync_copy(sgn_hbm, sgn_v)

    in_bufs = ((x0, si0), (x1, si1))
    out_bufs = ((o0, so0), (o1, so1))

    def in_slice(c):
        return x_hbm.at[pl.ds(pl.multiple_of(base + c * _R, _R), _R)]

    def out_slice(c, h):
        return out_hbm.at[pl.ds(base + c * _R + h * _H, _H)]

    pltpu.async_copy(in_slice(0), x0, si0)
    pltpu.async_copy(in_slice(1), x1, si1)
    pltpu.async_copy(o0, out_slice(0, 0), so0)
    pltpu.async_copy(o1, out_slice(0, 1), so1)

    row_splats = [jnp.full((16,), r, jnp.int32) for r in range(_R)]

    def compute(xb):
        @plsc.parallel_loop(0, _L, 1, unroll=4)
        def ibody(i):
            off0 = i * _JW
            for t in range(_NT):
                off = off0 + t * 16
                idxv = idx_v[pl.ds(off, 16)]
                sgnv = sgn_v[pl.ds(off, 16)]
                for r in range(_R):
                    v = plsc.load_gather(xb, [row_splats[r], idxv])
                    ob, _ = out_bufs[r // _H]
                    ob[r % _H, i, pl.ds(_JS[t], 16)] = v * sgnv

    @pl.loop(0, _NPAIR)
    def pair(cp):
        for b in range(2):
            xb, si = in_bufs[b]
            c = 2 * cp + b
            pltpu.make_async_copy(in_slice(c), xb, si).wait()
            for h in range(2):
                ob, so = out_bufs[h]
                pltpu.make_async_copy(ob, out_slice(c, h), so).wait()
            compute(xb)
            for h in range(2):
                ob, so = out_bufs[h]
                pltpu.async_copy(ob, out_slice(c, h), so)

            @pl.when(cp < _NPAIR - 1)
            def _next_in():
                pltpu.async_copy(in_slice(c + 2), xb, si)

    pltpu.make_async_copy(o0, out_slice(_NCHUNK - 1, 0), so0).wait()
    pltpu.make_async_copy(o1, out_slice(_NCHUNK - 1, 1), so1).wait()


@jax.jit
def kernel(rel_column):
    idx, sgn = _tables()
    mesh = plsc.VectorSubcoreMesh(core_axis_name="c", subcore_axis_name="s")
    call = pl.kernel(
        _body,
        out_type=jax.ShapeDtypeStruct((_B, 72, 128), jnp.float32),
        mesh=mesh,
        compiler_params=pltpu.CompilerParams(needs_layout_passes=False),
        scratch_types=[
            pltpu.VMEM((_TAB,), jnp.int32),
            pltpu.VMEM((_TAB,), jnp.float32),
            pltpu.VMEM((_R, _NCOL), jnp.float32),
            pltpu.VMEM((_R, _NCOL), jnp.float32),
            pltpu.VMEM((_H, 72, 128), jnp.float32),
            pltpu.VMEM((_H, 72, 128), jnp.float32),
            pltpu.SemaphoreType.DMA,
            pltpu.SemaphoreType.DMA,
            pltpu.SemaphoreType.DMA,
            pltpu.SemaphoreType.DMA,
        ],
    )
    padded = call(rel_column, idx, sgn)
    return padded[:, :_L, :_L]


# speedup vs baseline: 1.0826x; 1.0826x over previous
import functools

import numpy as np
import jax
import jax.numpy as jnp
from jax import lax
from jax.experimental import pallas as pl
from jax.experimental.pallas import tpu as pltpu
from jax.experimental.pallas import tpu_sc as plsc

_L = 68
_NCOL = _L * (_L - 1) // 2
_B = 16384
_NW = 32
_ROWS_PER_W = _B // _NW
_R = 8
_NCHUNK = _ROWS_PER_W // _R
_NPAIR = _NCHUNK // 2
_JS = (0, 16, 32, 48, 64)
_NT = len(_JS)
_JW = _NT * 16
_TAB = _L * _JW


def _tables():
    i = np.arange(_L)[:, None]
    jpos = np.concatenate([np.arange(js, js + 16) for js in _JS])[None, :]
    tri_i = (i * (i - 1)) // 2
    tri_j = (jpos * (jpos - 1)) // 2
    idx = np.where(i > jpos, tri_i + jpos, np.minimum(tri_j + i, _NCOL - 1))
    idx = np.where(i == jpos, 0, idx)
    sgn = np.where(i > jpos, 1.0, np.where(i < jpos, -1.0, 0.0))
    sgn = np.where(jpos >= _L, 0.0, sgn)
    return (jnp.asarray(idx.reshape(-1), dtype=jnp.int32),
            jnp.asarray(sgn.reshape(-1), dtype=jnp.float32))


_H = _R // 2


def _body(x_hbm, idx_hbm, sgn_hbm, out_hbm,
          idx_v, sgn_v, x0, x1, o0, o1, si0, si1, so0, so1):
    wid = lax.axis_index("s") * 2 + lax.axis_index("c")
    base = wid * _ROWS_PER_W
    pltpu.sync_copy(idx_hbm, idx_v)
    pltpu.sync_copy(sgn_hbm, sgn_v)

    in_bufs = ((x0, si0), (x1, si1))
    out_bufs = ((o0, so0), (o1, so1))

    def in_slice(c):
        return x_hbm.at[pl.ds(pl.multiple_of(base + c * _R, _R), _R)]

    def out_slice(c, h):
        return out_hbm.at[pl.ds(base + c * _R + h * _H, _H)]

    pltpu.async_copy(in_slice(0), x0, si0)
    pltpu.async_copy(in_slice(1), x1, si1)
    pltpu.async_copy(o0, out_slice(0, 0), so0)
    pltpu.async_copy(o1, out_slice(0, 1), so1)

    row_splats = [jnp.full((16,), r, jnp.int32) for r in range(_H)]

    def compute(xb, ob, h):
        @plsc.parallel_loop(0, _L, 1, unroll=4)
        def ibody(i):
            off0 = i * _JW
            for t in range(_NT):
                off = off0 + t * 16
                idxv = idx_v[pl.ds(off, 16)]
                sgnv = sgn_v[pl.ds(off, 16)]
                for r in range(_H):
                    v = plsc.load_gather(xb, [row_splats[r] + h * _H, idxv])
                    ob[r, i, pl.ds(_JS[t], 16)] = v * sgnv

    @pl.loop(0, _NPAIR)
    def pair(cp):
        for b in range(2):
            xb, si = in_bufs[b]
            c = 2 * cp + b
            pltpu.make_async_copy(in_slice(c), xb, si).wait()
            for h in range(2):
                ob, so = out_bufs[h]
                pltpu.make_async_copy(ob, out_slice(c, h), so).wait()
                compute(xb, ob, h)
                pltpu.async_copy(ob, out_slice(c, h), so)

            @pl.when(cp < _NPAIR - 1)
            def _next_in():
                pltpu.async_copy(in_slice(c + 2), xb, si)

    pltpu.make_async_copy(o0, out_slice(_NCHUNK - 1, 0), so0).wait()
    pltpu.make_async_copy(o1, out_slice(_NCHUNK - 1, 1), so1).wait()


@jax.jit
def kernel(rel_column):
    idx, sgn = _tables()
    mesh = plsc.VectorSubcoreMesh(core_axis_name="c", subcore_axis_name="s")
    call = pl.kernel(
        _body,
        out_type=jax.ShapeDtypeStruct((_B, 72, 128), jnp.float32),
        mesh=mesh,
        compiler_params=pltpu.CompilerParams(needs_layout_passes=False),
        scratch_types=[
            pltpu.VMEM((_TAB,), jnp.int32),
            pltpu.VMEM((_TAB,), jnp.float32),
            pltpu.VMEM((_R, _NCOL), jnp.float32),
            pltpu.VMEM((_R, _NCOL), jnp.float32),
            pltpu.VMEM((_H, 72, 128), jnp.float32),
            pltpu.VMEM((_H, 72, 128), jnp.float32),
            pltpu.SemaphoreType.DMA,
            pltpu.SemaphoreType.DMA,
            pltpu.SemaphoreType.DMA,
            pltpu.SemaphoreType.DMA,
        ],
    )
    padded = call(rel_column, idx, sgn)
    return padded[:, :_L, :_L]
